# baseline (speedup 1.0000x reference)
import functools

import jax
import jax.numpy as jnp
from jax import lax
from jax.experimental import pallas as pl
from jax.experimental.pallas import tpu as pltpu
from jax.experimental.pallas import tpu_sc as plsc

NUM_CORES = 2
NUM_SUBCORES = 16
NUM_WORKERS = NUM_CORES * NUM_SUBCORES
CHUNK = 128
TPAD = 128
BR = 8192


def _make_gather(B, V):
    b_per_w = B // NUM_WORKERS
    chunks_per_w = b_per_w // CHUNK

    mesh = plsc.VectorSubcoreMesh(core_axis_name="c", subcore_axis_name="s")

    @functools.partial(
        pl.kernel,
        mesh=mesh,
        out_type=jax.ShapeDtypeStruct((B, TPAD), jnp.float32),
        scratch_types=[
            pltpu.VMEM((b_per_w,), jnp.int32),
            pltpu.VMEM((b_per_w, TPAD), jnp.float32),
            pltpu.SemaphoreType.DMA,
            pltpu.SemaphoreType.DMA,
        ],
    )
    def k(idx_hbm, table_hbm, emb_hbm, idx_v, emb_v, sg, so):
        wid = lax.axis_index("s") * NUM_CORES + lax.axis_index("c")
        base = wid * b_per_w
        pltpu.sync_copy(idx_hbm.at[pl.ds(base, b_per_w)], idx_v)
        cps = [
            pltpu.async_copy(
                table_hbm.at[idx_v.at[pl.ds(j * CHUNK, CHUNK)]],
                emb_v.at[pl.ds(j * CHUNK, CHUNK)],
                sg,
            )
            for j in range(chunks_per_w)
        ]
        half = (chunks_per_w // 2) * CHUNK
        for cp in cps[: chunks_per_w // 2]:
            cp.wait()
        w0 = pltpu.async_copy(
            emb_v.at[pl.ds(0, half)], emb_hbm.at[pl.ds(base, half)], so
        )
        for cp in cps[chunks_per_w // 2:]:
            cp.wait()
        w1 = pltpu.async_copy(
            emb_v.at[pl.ds(half, b_per_w - half)],
            emb_hbm.at[pl.ds(base + half, b_per_w - half)],
            so,
        )
        w0.wait()
        w1.wait()

    return k


def _cond_t_body(cond_ref, out_ref):
    out_ref[...] = cond_ref[...].T


def _make_cond_t(B, CD, OUT_D):
    return pl.pallas_call(
        _cond_t_body,
        grid=(B // BR,),
        in_specs=[pl.BlockSpec((BR, CD), lambda i: (i, 0))],
        out_specs=pl.BlockSpec((CD, BR), lambda i: (0, i)),
        out_shape=jax.ShapeDtypeStruct((OUT_D, B), jnp.float32),
        compiler_params=pltpu.CompilerParams(
            dimension_semantics=("arbitrary",),
        ),
    )


def _emb_insert_body(emb_ref, acc_ref, out_ref):
    del acc_ref
    out_ref[...] = emb_ref[:, : out_ref.shape[0]].T


def _make_emb_insert(B, CD, D, OUT_D):
    assert CD % D == 0
    return pl.pallas_call(
        _emb_insert_body,
        grid=(B // BR,),
        in_specs=[
            pl.BlockSpec((BR, TPAD), lambda i: (i, 0)),
            pl.BlockSpec(memory_space=pl.ANY),
        ],
        out_specs=pl.BlockSpec((D, BR), lambda i: (CD // D, i)),
        out_shape=jax.ShapeDtypeStruct((OUT_D, B), jnp.float32),
        input_output_aliases={1: 0},
        compiler_params=pltpu.CompilerParams(
            dimension_semantics=("arbitrary",),
        ),
    )


def kernel(cond, task_ids, table):
    B, CD = cond.shape
    V, D = table.shape
    idx = task_ids.astype(jnp.int32)
    table_p = jnp.pad(table, ((0, 0), (0, TPAD - D)))
    emb = _make_gather(B, V)(idx, table_p)
    acc_t = _make_cond_t(B, CD, CD + D)(cond)
    out_t = _make_emb_insert(B, CD, D, CD + D)(emb, acc_t)
    return out_t.T


# speedup vs baseline: 1.1502x; 1.1502x over previous
import functools

import jax
import jax.numpy as jnp
from jax import lax
from jax.experimental import pallas as pl
from jax.experimental.pallas import tpu as pltpu
from jax.experimental.pallas import tpu_sc as plsc

NUM_CORES = 2
NUM_SUBCORES = 16
NUM_WORKERS = NUM_CORES * NUM_SUBCORES
CHUNK = 128
TPAD = 128
BR = 8192


def _make_gather(B, V):
    b_per_w = B // NUM_WORKERS
    chunks_per_w = b_per_w // CHUNK

    mesh = plsc.VectorSubcoreMesh(core_axis_name="c", subcore_axis_name="s")

    @functools.partial(
        pl.kernel,
        mesh=mesh,
        out_type=jax.ShapeDtypeStruct((B, TPAD), jnp.float32),
        scratch_types=[
            pltpu.VMEM((b_per_w,), jnp.int32),
            pltpu.VMEM((b_per_w, TPAD), jnp.float32),
            pltpu.VMEM_SHARED((V, TPAD), jnp.float32),
            pltpu.SemaphoreType.DMA,
            pltpu.SemaphoreType.DMA,
        ],
    )
    def k(idx_hbm, table_hbm, emb_hbm, idx_v, emb_v, table_s, sg, so):
        wid = lax.axis_index("s") * NUM_CORES + lax.axis_index("c")
        base = wid * b_per_w
        @pl.when(lax.axis_index("s") == 0)
        def _stage():
            pltpu.sync_copy(table_hbm, table_s)

        pltpu.sync_copy(idx_hbm.at[pl.ds(base, b_per_w)], idx_v)
        plsc.subcore_barrier()
        cps = [
            pltpu.async_copy(
                table_s.at[idx_v.at[pl.ds(j * CHUNK, CHUNK)]],
                emb_v.at[pl.ds(j * CHUNK, CHUNK)],
                sg,
            )
            for j in range(chunks_per_w)
        ]
        half = (chunks_per_w // 2) * CHUNK
        for cp in cps[: chunks_per_w // 2]:
            cp.wait()
        w0 = pltpu.async_copy(
            emb_v.at[pl.ds(0, half)], emb_hbm.at[pl.ds(base, half)], so
        )
        for cp in cps[chunks_per_w // 2:]:
            cp.wait()
        w1 = pltpu.async_copy(
            emb_v.at[pl.ds(half, b_per_w - half)],
            emb_hbm.at[pl.ds(base + half, b_per_w - half)],
            so,
        )
        w0.wait()
        w1.wait()

    return k


def _cond_t_body(cond_ref, out_ref):
    out_ref[...] = cond_ref[...].T


def _make_cond_t(B, CD, OUT_D):
    return pl.pallas_call(
        _cond_t_body,
        grid=(B // BR,),
        in_specs=[pl.BlockSpec((BR, CD), lambda i: (i, 0))],
        out_specs=pl.BlockSpec((CD, BR), lambda i: (0, i)),
        out_shape=jax.ShapeDtypeStruct((OUT_D, B), jnp.float32),
        compiler_params=pltpu.CompilerParams(
            dimension_semantics=("arbitrary",),
        ),
    )


def _emb_insert_body(emb_ref, acc_ref, out_ref):
    del acc_ref
    out_ref[...] = emb_ref[:, : out_ref.shape[0]].T


def _make_emb_insert(B, CD, D, OUT_D):
    assert CD % D == 0
    return pl.pallas_call(
        _emb_insert_body,
        grid=(B // BR,),
        in_specs=[
            pl.BlockSpec((BR, TPAD), lambda i: (i, 0)),
            pl.BlockSpec(memory_space=pl.ANY),
        ],
        out_specs=pl.BlockSpec((D, BR), lambda i: (CD // D, i)),
        out_shape=jax.ShapeDtypeStruct((OUT_D, B), jnp.float32),
        input_output_aliases={1: 0},
        compiler_params=pltpu.CompilerParams(
            dimension_semantics=("arbitrary",),
        ),
    )


def kernel(cond, task_ids, table):
    B, CD = cond.shape
    V, D = table.shape
    idx = task_ids.astype(jnp.int32)
    table_p = jnp.pad(table, ((0, 0), (0, TPAD - D)))
    emb = _make_gather(B, V)(idx, table_p)
    acc_t = _make_cond_t(B, CD, CD + D)(cond)
    out_t = _make_emb_insert(B, CD, D, CD + D)(emb, acc_t)
    return out_t.T


# speedup vs baseline: 1.1600x; 1.0086x over previous
import functools

import jax
import jax.numpy as jnp
from jax import lax
from jax.experimental import pallas as pl
from jax.experimental.pallas import tpu as pltpu
from jax.experimental.pallas import tpu_sc as plsc

NUM_CORES = 2
NUM_SUBCORES = 16
NUM_WORKERS = NUM_CORES * NUM_SUBCORES
CHUNK = 128
TPAD = 128
BR = 8192


def _make_gather(B, V):
    b_per_w = B // NUM_WORKERS
    chunks_per_w = b_per_w // CHUNK

    mesh = plsc.VectorSubcoreMesh(core_axis_name="c", subcore_axis_name="s")

    @functools.partial(
        pl.kernel,
        mesh=mesh,
        out_type=jax.ShapeDtypeStruct((B, 64), jnp.float32),
        scratch_types=[
            pltpu.VMEM((b_per_w,), jnp.int32),
            pltpu.VMEM((b_per_w, 64), jnp.float32),
            pltpu.VMEM_SHARED((V, 64), jnp.float32),
            pltpu.SemaphoreType.DMA,
            pltpu.SemaphoreType.DMA,
        ],
    )
    def k(idx_hbm, table_hbm, emb_hbm, idx_v, emb_v, table_s, sg, so):
        wid = lax.axis_index("s") * NUM_CORES + lax.axis_index("c")
        base = wid * b_per_w
        @pl.when(lax.axis_index("s") == 0)
        def _stage():
            pltpu.sync_copy(table_hbm, table_s)

        pltpu.sync_copy(idx_hbm.at[pl.ds(base, b_per_w)], idx_v)
        plsc.subcore_barrier()
        cps = [
            pltpu.async_copy(
                table_s.at[idx_v.at[pl.ds(j * CHUNK, CHUNK)]],
                emb_v.at[pl.ds(j * CHUNK, CHUNK)],
                sg,
            )
            for j in range(chunks_per_w)
        ]
        half = (chunks_per_w // 2) * CHUNK
        for cp in cps[: chunks_per_w // 2]:
            cp.wait()
        w0 = pltpu.async_copy(
            emb_v.at[pl.ds(0, half)], emb_hbm.at[pl.ds(base, half)], so
        )
        for cp in cps[chunks_per_w // 2:]:
            cp.wait()
        w1 = pltpu.async_copy(
            emb_v.at[pl.ds(half, b_per_w - half)],
            emb_hbm.at[pl.ds(base + half, b_per_w - half)],
            so,
        )
        w0.wait()
        w1.wait()

    return k


def _cond_t_body(cond_ref, out_ref):
    out_ref[...] = cond_ref[...].T


def _make_cond_t(B, CD, OUT_D):
    return pl.pallas_call(
        _cond_t_body,
        grid=(B // BR,),
        in_specs=[pl.BlockSpec((BR, CD), lambda i: (i, 0))],
        out_specs=pl.BlockSpec((CD, BR), lambda i: (0, i)),
        out_shape=jax.ShapeDtypeStruct((OUT_D, B), jnp.float32),
        compiler_params=pltpu.CompilerParams(
            dimension_semantics=("arbitrary",),
        ),
    )


def _emb_insert_body(emb_ref, acc_ref, out_ref):
    del acc_ref
    out_ref[...] = emb_ref[...].T


def _make_emb_insert(B, CD, D, OUT_D):
    assert CD % D == 0
    return pl.pallas_call(
        _emb_insert_body,
        grid=(B // BR,),
        in_specs=[
            pl.BlockSpec((BR, D), lambda i: (i, 0)),
            pl.BlockSpec(memory_space=pl.ANY),
        ],
        out_specs=pl.BlockSpec((D, BR), lambda i: (CD // D, i)),
        out_shape=jax.ShapeDtypeStruct((OUT_D, B), jnp.float32),
        input_output_aliases={1: 0},
        compiler_params=pltpu.CompilerParams(
            dimension_semantics=("arbitrary",),
        ),
    )


def kernel(cond, task_ids, table):
    B, CD = cond.shape
    V, D = table.shape
    idx = task_ids.astype(jnp.int32)
    emb = _make_gather(B, V)(idx, table)
    acc_t = _make_cond_t(B, CD, CD + D)(cond)
    out_t = _make_emb_insert(B, CD, D, CD + D)(emb, acc_t)
    return out_t.T
